# Initial kernel scaffold; baseline (speedup 1.0000x reference)
import jax
import jax.numpy as jnp
from jax import lax
from jax.experimental import pallas as pl
from jax.experimental.pallas import tpu as pltpu
from jax.experimental.pallas import tpu_sc as plsc

_N = 10000
_D = 128
_H = 64
_E = 320000
_CLS = 10
_EPS = 1e-12

_C = 128
_NCH = _E // _C
_NS = 16
_T = -(-_NCH // _NS)
_RPT = _N // _NS
_ZR = 125

_BN = 1000
_NB = _N // _BN


_DN = (((1,), (1,)), ((), ()))


def _dot_t(a, b):
    return lax.dot_general(a, b, _DN, preferred_element_type=jnp.float32)


def _tc_pre_body(x_ref, wmi_ref, wmo_ref, ws_ref,
                 hmi0_ref, hmi1_ref, hmo0_ref, hmo1_ref, sh_ref):
    xb = x_ref[...]
    hmi0_ref[...] = _dot_t(xb, wmi_ref[0:_H, :])
    hmi1_ref[...] = _dot_t(xb, wmi_ref[_H:_D, :])
    hmo0_ref[...] = _dot_t(xb, wmo_ref[0:_H, :])
    hmo1_ref[...] = _dot_t(xb, wmo_ref[_H:_D, :])
    sh_ref[...] = _dot_t(xb, ws_ref[...])


def _tc_pre(x, wmi, wmo, ws):
    f32 = jnp.float32
    row = pl.BlockSpec((_BN, _D), lambda i: (i, 0))
    half = pl.BlockSpec((_BN, _H), lambda i: (i, 0))
    full_w = pl.BlockSpec((_D, _D), lambda i: (0, 0))
    return pl.pallas_call(
        _tc_pre_body,
        grid=(_NB,),
        in_specs=[row, full_w, full_w, full_w],
        out_specs=[half, half, half, half, row],
        out_shape=[jax.ShapeDtypeStruct((_N, _H), f32)] * 4
        + [jax.ShapeDtypeStruct((_N, _D), f32)],
    )(x, wmi, wmo, ws)


def _combine(pin0, pin1, pout0, pout1, sh_ref, bmi, bmo, bsi, bso, cin, cout):
    sh = sh_ref[...]
    pin = jnp.concatenate([pin0[...], pin1[...]], axis=1)
    pout = jnp.concatenate([pout0[...], pout1[...]], axis=1)
    ic = pin + bmi[...] + sh + bsi[...]
    oc = pout + bmo[...] + sh + bso[...]
    return jnp.tanh(cin[...] * ic + cout[...] * oc)


def _tc_mid_body(pin0, pin1, pout0, pout1, sh_ref, bmi, bmo, bsi, bso,
                 cin, cout, wmi_ref, wmo_ref, ws_ref,
                 hmi0_ref, hmi1_ref, hmo0_ref, hmo1_ref, sh_o):
    h = _combine(pin0, pin1, pout0, pout1, sh_ref, bmi, bmo, bsi, bso, cin, cout)
    hmi0_ref[...] = _dot_t(h, wmi_ref[0:_H, :])
    hmi1_ref[...] = _dot_t(h, wmi_ref[_H:_D, :])
    hmo0_ref[...] = _dot_t(h, wmo_ref[0:_H, :])
    hmo1_ref[...] = _dot_t(h, wmo_ref[_H:_D, :])
    sh_o[...] = _dot_t(h, ws_ref[...])


def _tc_mid(pin0, pin1, pout0, pout1, sh, bmi, bmo, bsi, bso, cin, cout,
            wmi, wmo, ws):
    f32 = jnp.float32
    row = pl.BlockSpec((_BN, _D), lambda i: (i, 0))
    half = pl.BlockSpec((_BN, _H), lambda i: (i, 0))
    bias = pl.BlockSpec((1, _D), lambda i: (0, 0))
    cvec = pl.BlockSpec((_BN, 1), lambda i: (i, 0))
    full_w = pl.BlockSpec((_D, _D), lambda i: (0, 0))
    return pl.pallas_call(
        _tc_mid_body,
        grid=(_NB,),
        in_specs=[half, half, half, half, row, bias, bias, bias, bias,
                  cvec, cvec, full_w, full_w, full_w],
        out_specs=[half, half, half, half, row],
        out_shape=[jax.ShapeDtypeStruct((_N, _H), f32)] * 4
        + [jax.ShapeDtypeStruct((_N, _D), f32)],
    )(pin0, pin1, pout0, pout1, sh, bmi, bmo, bsi, bso, cin, cout, wmi, wmo, ws)


def _tc_post_body(pin0, pin1, pout0, pout1, sh_ref, bmi, bmo, bsi, bso,
                  cin, cout, wdec_ref, bdec_ref, logp_ref, emb_ref):
    h = _combine(pin0, pin1, pout0, pout1, sh_ref, bmi, bmo, bsi, bso, cin, cout)
    logits = _dot_t(h, wdec_ref[...]) + bdec_ref[...]
    m = jnp.max(logits, axis=-1, keepdims=True)
    e = jnp.exp(logits - m)
    lse = jnp.log(jnp.sum(e, axis=-1, keepdims=True)) + m
    logp_ref[...] = logits - lse
    nrm = jnp.sqrt(jnp.sum(h * h, axis=-1, keepdims=True))
    emb_ref[...] = h / (nrm + _EPS)


def _tc_post(pin0, pin1, pout0, pout1, sh, bmi, bmo, bsi, bso, cin, cout,
             wdec, bdec):
    f32 = jnp.float32
    row = pl.BlockSpec((_BN, _D), lambda i: (i, 0))
    half = pl.BlockSpec((_BN, _H), lambda i: (i, 0))
    bias = pl.BlockSpec((1, _D), lambda i: (0, 0))
    cvec = pl.BlockSpec((_BN, 1), lambda i: (i, 0))
    return pl.pallas_call(
        _tc_post_body,
        grid=(_NB,),
        in_specs=[half, half, half, half, row, bias, bias, bias, bias,
                  cvec, cvec,
                  pl.BlockSpec((_CLS, _D), lambda i: (0, 0)),
                  pl.BlockSpec((1, _CLS), lambda i: (0, 0))],
        out_specs=[pl.BlockSpec((_BN, _CLS), lambda i: (i, 0)), row],
        out_shape=[jax.ShapeDtypeStruct((_N, _CLS), f32),
                   jax.ShapeDtypeStruct((_N, _D), f32)],
    )(pin0, pin1, pout0, pout1, sh, bmi, bmo, bsi, bso, cin, cout, wdec, bdec)


def _sc_body(hmi0, hmi1, hmo0, hmo1, si, di, wi, so, do_, wo,
             pin0, pin1, pout0, pout1,
             acc_in, acc_out, srcv, dstv, wv, rows, zbuf, sem):
    c = lax.axis_index("c")
    s = lax.axis_index("s")

    zeros16 = jnp.zeros((16,), jnp.float32)

    def zinit(i, carry):
        for k4 in range(_H // 16):
            zbuf[i, k4 * 16:(k4 + 1) * 16] = zeros16
        return carry

    lax.fori_loop(0, _ZR, zinit, 0)
    base_r = s * _RPT
    for kk in range(_RPT // _ZR):
        pltpu.sync_copy(zbuf, acc_in.at[pl.ds(base_r + kk * _ZR, _ZR)])
        pltpu.sync_copy(zbuf, acc_out.at[pl.ds(base_r + kk * _ZR, _ZR)])
    plsc.subcore_barrier()

    def run_edges(src_h, dst_h, w_h, tab_h, acc):
        def chunk(jj, carry):
            j = jj * _NS + s

            @pl.when(j < _NCH)
            def _():
                base = j * _C
                pltpu.sync_copy(src_h.at[pl.ds(base, _C)], srcv)
                pltpu.sync_copy(dst_h.at[pl.ds(base, _C)], dstv)
                pltpu.sync_copy(w_h.at[pl.ds(base, _C)], wv)
                pltpu.async_copy(tab_h.at[srcv], rows, sem).wait()

                def scale(i, cc):
                    wsc = wv[i]
                    for k4 in range(_H // 16):
                        sl = pl.ds(k4 * 16, 16)
                        rows[i, sl] = rows[i, sl] * wsc
                    return cc

                lax.fori_loop(0, _C, scale, 0)
                pltpu.sync_copy(rows, acc.at[dstv], add=True)

            return carry

        lax.fori_loop(0, _T, chunk, 0)

    @pl.when(c == 0)
    def _():
        run_edges(si, di, wi, hmi0, acc_in)
        run_edges(so, do_, wo, hmo0, acc_out)

    @pl.when(c == 1)
    def _():
        run_edges(si, di, wi, hmi1, acc_in)
        run_edges(so, do_, wo, hmo1, acc_out)

    plsc.subcore_barrier()

    @pl.when(c == 0)
    def _():
        pltpu.sync_copy(acc_in.at[pl.ds(base_r, _RPT)], pin0.at[pl.ds(base_r, _RPT)])
        pltpu.sync_copy(acc_out.at[pl.ds(base_r, _RPT)], pout0.at[pl.ds(base_r, _RPT)])

    @pl.when(c == 1)
    def _():
        pltpu.sync_copy(acc_in.at[pl.ds(base_r, _RPT)], pin1.at[pl.ds(base_r, _RPT)])
        pltpu.sync_copy(acc_out.at[pl.ds(base_r, _RPT)], pout1.at[pl.ds(base_r, _RPT)])


def _sc_propagate(hmi0, hmi1, hmo0, hmo1, si, di, wi, so, do_, wo):
    f32 = jnp.float32
    mesh = plsc.VectorSubcoreMesh(core_axis_name="c", subcore_axis_name="s")
    kfn = pl.kernel(
        _sc_body,
        out_type=[jax.ShapeDtypeStruct((_N, _H), f32)] * 4,
        mesh=mesh,
        scratch_types=[
            pltpu.VMEM_SHARED((_N, _H), f32),
            pltpu.VMEM_SHARED((_N, _H), f32),
            pltpu.VMEM((_C,), jnp.int32),
            pltpu.VMEM((_C,), jnp.int32),
            pltpu.VMEM((_C,), f32),
            pltpu.VMEM((_C, _H), f32),
            pltpu.VMEM((_ZR, _H), f32),
            pltpu.SemaphoreType.DMA,
        ],
    )
    return kfn(hmi0, hmi1, hmo0, hmo1, si, di, wi, so, do_, wo)


def kernel(x, edge_index_in, edge_weight_in, edge_index_out, edge_weight_out,
           Wmi0, Wmo0, Ws0, bmi0, bmo0, bsi0, bso0, Cin0, Cout0,
           Wmi1, Wmo1, Ws1, bmi1, bmo1, bsi1, bso1, Cin1, Cout1,
           W_dec, b_dec):
    si = edge_index_in[0]
    ti = edge_index_in[1]
    so = edge_index_out[0]
    to = edge_index_out[1]

    bmi0r, bmo0r = bmi0.reshape(1, _D), bmo0.reshape(1, _D)
    bsi0r, bso0r = bsi0.reshape(1, _D), bso0.reshape(1, _D)
    bmi1r, bmo1r = bmi1.reshape(1, _D), bmo1.reshape(1, _D)
    bsi1r, bso1r = bsi1.reshape(1, _D), bso1.reshape(1, _D)
    bdecr = b_dec.reshape(1, _CLS)

    hmi0a, hmi0b, hmo0a, hmo0b, sh0 = _tc_pre(x, Wmi0, Wmo0, Ws0)
    pin0a, pin0b, pout0a, pout0b = _sc_propagate(
        hmi0a, hmi0b, hmo0a, hmo0b, si, ti, edge_weight_in, so, to,
        edge_weight_out)
    hmi1a, hmi1b, hmo1a, hmo1b, sh1 = _tc_mid(
        pin0a, pin0b, pout0a, pout0b, sh0, bmi0r, bmo0r, bsi0r, bso0r,
        Cin0, Cout0, Wmi1, Wmo1, Ws1)
    pin1a, pin1b, pout1a, pout1b = _sc_propagate(
        hmi1a, hmi1b, hmo1a, hmo1b, si, ti, edge_weight_in, so, to,
        edge_weight_out)
    logp, emb = _tc_post(
        pin1a, pin1b, pout1a, pout1b, sh1, bmi1r, bmo1r, bsi1r, bso1r,
        Cin1, Cout1, W_dec, bdecr)
    return (logp, emb)


# speedup vs baseline: 1.6617x; 1.6617x over previous
import jax
import jax.numpy as jnp
from jax import lax
from jax.experimental import pallas as pl
from jax.experimental.pallas import tpu as pltpu
from jax.experimental.pallas import tpu_sc as plsc

_N = 10000
_D = 128
_H = 64
_E = 320000
_CLS = 10
_EPS = 1e-12

_C = 128
_NCH = _E // _C
_NS = 16
_T = -(-_NCH // _NS)
_RPT = _N // _NS
_ZR = 125

_BN = 1000
_NB = _N // _BN


_DN = (((1,), (1,)), ((), ()))


def _dot_t(a, b):
    return lax.dot_general(a, b, _DN, preferred_element_type=jnp.float32)


def _tc_pre_body(x_ref, wmi_ref, wmo_ref, ws_ref,
                 hmi0_ref, hmi1_ref, hmo0_ref, hmo1_ref, sh_ref):
    xb = x_ref[...]
    hmi0_ref[...] = _dot_t(xb, wmi_ref[0:_H, :])
    hmi1_ref[...] = _dot_t(xb, wmi_ref[_H:_D, :])
    hmo0_ref[...] = _dot_t(xb, wmo_ref[0:_H, :])
    hmo1_ref[...] = _dot_t(xb, wmo_ref[_H:_D, :])
    sh_ref[...] = _dot_t(xb, ws_ref[...])


def _tc_pre(x, wmi, wmo, ws):
    f32 = jnp.float32
    row = pl.BlockSpec((_BN, _D), lambda i: (i, 0))
    half = pl.BlockSpec((_BN, _H), lambda i: (i, 0))
    full_w = pl.BlockSpec((_D, _D), lambda i: (0, 0))
    return pl.pallas_call(
        _tc_pre_body,
        grid=(_NB,),
        in_specs=[row, full_w, full_w, full_w],
        out_specs=[half, half, half, half, row],
        out_shape=[jax.ShapeDtypeStruct((_N, _H), f32)] * 4
        + [jax.ShapeDtypeStruct((_N, _D), f32)],
    )(x, wmi, wmo, ws)


def _combine(pin0, pin1, pout0, pout1, sh_ref, bmi, bmo, bsi, bso, cin, cout):
    sh = sh_ref[...]
    pin = jnp.concatenate([pin0[...], pin1[...]], axis=1)
    pout = jnp.concatenate([pout0[...], pout1[...]], axis=1)
    ic = pin + bmi[...] + sh + bsi[...]
    oc = pout + bmo[...] + sh + bso[...]
    return jnp.tanh(cin[...] * ic + cout[...] * oc)


def _tc_mid_body(pin0, pin1, pout0, pout1, sh_ref, bmi, bmo, bsi, bso,
                 cin, cout, wmi_ref, wmo_ref, ws_ref,
                 hmi0_ref, hmi1_ref, hmo0_ref, hmo1_ref, sh_o):
    h = _combine(pin0, pin1, pout0, pout1, sh_ref, bmi, bmo, bsi, bso, cin, cout)
    hmi0_ref[...] = _dot_t(h, wmi_ref[0:_H, :])
    hmi1_ref[...] = _dot_t(h, wmi_ref[_H:_D, :])
    hmo0_ref[...] = _dot_t(h, wmo_ref[0:_H, :])
    hmo1_ref[...] = _dot_t(h, wmo_ref[_H:_D, :])
    sh_o[...] = _dot_t(h, ws_ref[...])


def _tc_mid(pin0, pin1, pout0, pout1, sh, bmi, bmo, bsi, bso, cin, cout,
            wmi, wmo, ws):
    f32 = jnp.float32
    row = pl.BlockSpec((_BN, _D), lambda i: (i, 0))
    half = pl.BlockSpec((_BN, _H), lambda i: (i, 0))
    bias = pl.BlockSpec((1, _D), lambda i: (0, 0))
    cvec = pl.BlockSpec((_BN, 1), lambda i: (i, 0))
    full_w = pl.BlockSpec((_D, _D), lambda i: (0, 0))
    return pl.pallas_call(
        _tc_mid_body,
        grid=(_NB,),
        in_specs=[half, half, half, half, row, bias, bias, bias, bias,
                  cvec, cvec, full_w, full_w, full_w],
        out_specs=[half, half, half, half, row],
        out_shape=[jax.ShapeDtypeStruct((_N, _H), f32)] * 4
        + [jax.ShapeDtypeStruct((_N, _D), f32)],
    )(pin0, pin1, pout0, pout1, sh, bmi, bmo, bsi, bso, cin, cout, wmi, wmo, ws)


def _tc_post_body(pin0, pin1, pout0, pout1, sh_ref, bmi, bmo, bsi, bso,
                  cin, cout, wdec_ref, bdec_ref, logp_ref, emb_ref):
    h = _combine(pin0, pin1, pout0, pout1, sh_ref, bmi, bmo, bsi, bso, cin, cout)
    logits = _dot_t(h, wdec_ref[...]) + bdec_ref[...]
    m = jnp.max(logits, axis=-1, keepdims=True)
    e = jnp.exp(logits - m)
    lse = jnp.log(jnp.sum(e, axis=-1, keepdims=True)) + m
    logp_ref[...] = logits - lse
    nrm = jnp.sqrt(jnp.sum(h * h, axis=-1, keepdims=True))
    emb_ref[...] = h / (nrm + _EPS)


def _tc_post(pin0, pin1, pout0, pout1, sh, bmi, bmo, bsi, bso, cin, cout,
             wdec, bdec):
    f32 = jnp.float32
    row = pl.BlockSpec((_BN, _D), lambda i: (i, 0))
    half = pl.BlockSpec((_BN, _H), lambda i: (i, 0))
    bias = pl.BlockSpec((1, _D), lambda i: (0, 0))
    cvec = pl.BlockSpec((_BN, 1), lambda i: (i, 0))
    return pl.pallas_call(
        _tc_post_body,
        grid=(_NB,),
        in_specs=[half, half, half, half, row, bias, bias, bias, bias,
                  cvec, cvec,
                  pl.BlockSpec((_CLS, _D), lambda i: (0, 0)),
                  pl.BlockSpec((1, _CLS), lambda i: (0, 0))],
        out_specs=[pl.BlockSpec((_BN, _CLS), lambda i: (i, 0)), row],
        out_shape=[jax.ShapeDtypeStruct((_N, _CLS), f32),
                   jax.ShapeDtypeStruct((_N, _D), f32)],
    )(pin0, pin1, pout0, pout1, sh, bmi, bmo, bsi, bso, cin, cout, wdec, bdec)


def _sc_body(hmi0, hmi1, hmo0, hmo1, si, di, wi, so, do_, wo,
             pin0, pin1, pout0, pout1,
             acc_in, acc_out, srcv, dstv, wv, rows, zbuf, sem):
    c = lax.axis_index("c")
    s = lax.axis_index("s")

    zeros16 = jnp.zeros((16,), jnp.float32)

    def zinit(i, carry):
        for k4 in range(_H // 16):
            zbuf[i, k4 * 16:(k4 + 1) * 16] = zeros16
        return carry

    lax.fori_loop(0, _ZR, zinit, 0)
    base_r = s * _RPT
    for kk in range(_RPT // _ZR):
        pltpu.sync_copy(zbuf, acc_in.at[pl.ds(base_r + kk * _ZR, _ZR)])
        pltpu.sync_copy(zbuf, acc_out.at[pl.ds(base_r + kk * _ZR, _ZR)])
    plsc.subcore_barrier()

    def run_edges(src_h, dst_h, w_h, tab_h, acc):
        def chunk(jj, carry):
            j = jj * _NS + s

            @pl.when(j < _NCH)
            def _():
                base = j * _C
                pltpu.sync_copy(src_h.at[pl.ds(base, _C)], srcv)
                pltpu.sync_copy(dst_h.at[pl.ds(base, _C)], dstv)
                pltpu.sync_copy(w_h.at[pl.ds(base, _C)], wv)
                pltpu.async_copy(tab_h.at[srcv], rows, sem).wait()

                def scale(i16, cc):
                    w16 = wv[pl.ds(i16 * 16, 16)]
                    for e in range(16):
                        wsc = w16[e]
                        r = i16 * 16 + e
                        for k4 in range(_H // 16):
                            sl = pl.ds(k4 * 16, 16)
                            rows[r, sl] = rows[r, sl] * wsc
                    return cc

                lax.fori_loop(0, _C // 16, scale, 0)
                pltpu.sync_copy(rows, acc.at[dstv], add=True)

            return carry

        lax.fori_loop(0, _T, chunk, 0)

    @pl.when(c == 0)
    def _():
        run_edges(si, di, wi, hmi0, acc_in)
        run_edges(so, do_, wo, hmo0, acc_out)

    @pl.when(c == 1)
    def _():
        run_edges(si, di, wi, hmi1, acc_in)
        run_edges(so, do_, wo, hmo1, acc_out)

    plsc.subcore_barrier()

    @pl.when(c == 0)
    def _():
        pltpu.sync_copy(acc_in.at[pl.ds(base_r, _RPT)], pin0.at[pl.ds(base_r, _RPT)])
        pltpu.sync_copy(acc_out.at[pl.ds(base_r, _RPT)], pout0.at[pl.ds(base_r, _RPT)])

    @pl.when(c == 1)
    def _():
        pltpu.sync_copy(acc_in.at[pl.ds(base_r, _RPT)], pin1.at[pl.ds(base_r, _RPT)])
        pltpu.sync_copy(acc_out.at[pl.ds(base_r, _RPT)], pout1.at[pl.ds(base_r, _RPT)])


def _sc_propagate(hmi0, hmi1, hmo0, hmo1, si, di, wi, so, do_, wo):
    f32 = jnp.float32
    mesh = plsc.VectorSubcoreMesh(core_axis_name="c", subcore_axis_name="s")
    kfn = pl.kernel(
        _sc_body,
        out_type=[jax.ShapeDtypeStruct((_N, _H), f32)] * 4,
        mesh=mesh,
        compiler_params=pltpu.CompilerParams(use_tc_tiling_on_sc=False),
        scratch_types=[
            pltpu.VMEM_SHARED((_N, _H), f32),
            pltpu.VMEM_SHARED((_N, _H), f32),
            pltpu.VMEM((_C,), jnp.int32),
            pltpu.VMEM((_C,), jnp.int32),
            pltpu.VMEM((_C,), f32),
            pltpu.VMEM((_C, _H), f32),
            pltpu.VMEM((_ZR, _H), f32),
            pltpu.SemaphoreType.DMA,
        ],
    )
    return kfn(hmi0, hmi1, hmo0, hmo1, si, di, wi, so, do_, wo)


def kernel(x, edge_index_in, edge_weight_in, edge_index_out, edge_weight_out,
           Wmi0, Wmo0, Ws0, bmi0, bmo0, bsi0, bso0, Cin0, Cout0,
           Wmi1, Wmo1, Ws1, bmi1, bmo1, bsi1, bso1, Cin1, Cout1,
           W_dec, b_dec):
    si = edge_index_in[0]
    ti = edge_index_in[1]
    so = edge_index_out[0]
    to = edge_index_out[1]

    bmi0r, bmo0r = bmi0.reshape(1, _D), bmo0.reshape(1, _D)
    bsi0r, bso0r = bsi0.reshape(1, _D), bso0.reshape(1, _D)
    bmi1r, bmo1r = bmi1.reshape(1, _D), bmo1.reshape(1, _D)
    bsi1r, bso1r = bsi1.reshape(1, _D), bso1.reshape(1, _D)
    bdecr = b_dec.reshape(1, _CLS)

    hmi0a, hmi0b, hmo0a, hmo0b, sh0 = _tc_pre(x, Wmi0, Wmo0, Ws0)
    pin0a, pin0b, pout0a, pout0b = _sc_propagate(
        hmi0a, hmi0b, hmo0a, hmo0b, si, ti, edge_weight_in, so, to,
        edge_weight_out)
    hmi1a, hmi1b, hmo1a, hmo1b, sh1 = _tc_mid(
        pin0a, pin0b, pout0a, pout0b, sh0, bmi0r, bmo0r, bsi0r, bso0r,
        Cin0, Cout0, Wmi1, Wmo1, Ws1)
    pin1a, pin1b, pout1a, pout1b = _sc_propagate(
        hmi1a, hmi1b, hmo1a, hmo1b, si, ti, edge_weight_in, so, to,
        edge_weight_out)
    logp, emb = _tc_post(
        pin1a, pin1b, pout1a, pout1b, sh1, bmi1r, bmo1r, bsi1r, bso1r,
        Cin1, Cout1, W_dec, bdecr)
    return (logp, emb)


# speedup vs baseline: 5.2406x; 3.1538x over previous
import jax
import jax.numpy as jnp
from jax import lax
from jax.experimental import pallas as pl
from jax.experimental.pallas import tpu as pltpu
from jax.experimental.pallas import tpu_sc as plsc

_N = 10000
_D = 128
_H = 64
_E = 320000
_CLS = 10
_EPS = 1e-12

_C = 128
_NCH = _E // _C
_NS = 16
_T = -(-_NCH // _NS)
_RPT = _N // _NS
_ZR = 125

_BN = 1000
_NB = _N // _BN


_DN = (((1,), (1,)), ((), ()))


def _dot_t(a, b):
    return lax.dot_general(a, b, _DN, preferred_element_type=jnp.float32)


def _tc_pre_body(x_ref, wmi_ref, wmo_ref, ws_ref,
                 hmi0_ref, hmi1_ref, hmo0_ref, hmo1_ref, sh_ref):
    xb = x_ref[...]
    hmi0_ref[...] = _dot_t(xb, wmi_ref[0:_H, :])
    hmi1_ref[...] = _dot_t(xb, wmi_ref[_H:_D, :])
    hmo0_ref[...] = _dot_t(xb, wmo_ref[0:_H, :])
    hmo1_ref[...] = _dot_t(xb, wmo_ref[_H:_D, :])
    sh_ref[...] = _dot_t(xb, ws_ref[...])


def _tc_pre(x, wmi, wmo, ws):
    f32 = jnp.float32
    row = pl.BlockSpec((_BN, _D), lambda i: (i, 0))
    half = pl.BlockSpec((_BN, _H), lambda i: (i, 0))
    full_w = pl.BlockSpec((_D, _D), lambda i: (0, 0))
    return pl.pallas_call(
        _tc_pre_body,
        grid=(_NB,),
        in_specs=[row, full_w, full_w, full_w],
        out_specs=[half, half, half, half, row],
        out_shape=[jax.ShapeDtypeStruct((_N, _H), f32)] * 4
        + [jax.ShapeDtypeStruct((_N, _D), f32)],
    )(x, wmi, wmo, ws)


def _combine(pin0, pin1, pout0, pout1, sh_ref, bmi, bmo, bsi, bso, cin, cout):
    sh = sh_ref[...]
    pin = jnp.concatenate([pin0[...], pin1[...]], axis=1)
    pout = jnp.concatenate([pout0[...], pout1[...]], axis=1)
    ic = pin + bmi[...] + sh + bsi[...]
    oc = pout + bmo[...] + sh + bso[...]
    return jnp.tanh(cin[...] * ic + cout[...] * oc)


def _tc_mid_body(pin0, pin1, pout0, pout1, sh_ref, bmi, bmo, bsi, bso,
                 cin, cout, wmi_ref, wmo_ref, ws_ref,
                 hmi0_ref, hmi1_ref, hmo0_ref, hmo1_ref, sh_o):
    h = _combine(pin0, pin1, pout0, pout1, sh_ref, bmi, bmo, bsi, bso, cin, cout)
    hmi0_ref[...] = _dot_t(h, wmi_ref[0:_H, :])
    hmi1_ref[...] = _dot_t(h, wmi_ref[_H:_D, :])
    hmo0_ref[...] = _dot_t(h, wmo_ref[0:_H, :])
    hmo1_ref[...] = _dot_t(h, wmo_ref[_H:_D, :])
    sh_o[...] = _dot_t(h, ws_ref[...])


def _tc_mid(pin0, pin1, pout0, pout1, sh, bmi, bmo, bsi, bso, cin, cout,
            wmi, wmo, ws):
    f32 = jnp.float32
    row = pl.BlockSpec((_BN, _D), lambda i: (i, 0))
    half = pl.BlockSpec((_BN, _H), lambda i: (i, 0))
    bias = pl.BlockSpec((1, _D), lambda i: (0, 0))
    cvec = pl.BlockSpec((_BN, 1), lambda i: (i, 0))
    full_w = pl.BlockSpec((_D, _D), lambda i: (0, 0))
    return pl.pallas_call(
        _tc_mid_body,
        grid=(_NB,),
        in_specs=[half, half, half, half, row, bias, bias, bias, bias,
                  cvec, cvec, full_w, full_w, full_w],
        out_specs=[half, half, half, half, row],
        out_shape=[jax.ShapeDtypeStruct((_N, _H), f32)] * 4
        + [jax.ShapeDtypeStruct((_N, _D), f32)],
    )(pin0, pin1, pout0, pout1, sh, bmi, bmo, bsi, bso, cin, cout, wmi, wmo, ws)


def _tc_post_body(pin0, pin1, pout0, pout1, sh_ref, bmi, bmo, bsi, bso,
                  cin, cout, wdec_ref, bdec_ref, logp_ref, emb_ref):
    h = _combine(pin0, pin1, pout0, pout1, sh_ref, bmi, bmo, bsi, bso, cin, cout)
    logits = _dot_t(h, wdec_ref[...]) + bdec_ref[...]
    m = jnp.max(logits, axis=-1, keepdims=True)
    e = jnp.exp(logits - m)
    lse = jnp.log(jnp.sum(e, axis=-1, keepdims=True)) + m
    logp_ref[...] = logits - lse
    nrm = jnp.sqrt(jnp.sum(h * h, axis=-1, keepdims=True))
    emb_ref[...] = h / (nrm + _EPS)


def _tc_post(pin0, pin1, pout0, pout1, sh, bmi, bmo, bsi, bso, cin, cout,
             wdec, bdec):
    f32 = jnp.float32
    row = pl.BlockSpec((_BN, _D), lambda i: (i, 0))
    half = pl.BlockSpec((_BN, _H), lambda i: (i, 0))
    bias = pl.BlockSpec((1, _D), lambda i: (0, 0))
    cvec = pl.BlockSpec((_BN, 1), lambda i: (i, 0))
    return pl.pallas_call(
        _tc_post_body,
        grid=(_NB,),
        in_specs=[half, half, half, half, row, bias, bias, bias, bias,
                  cvec, cvec,
                  pl.BlockSpec((_CLS, _D), lambda i: (0, 0)),
                  pl.BlockSpec((1, _CLS), lambda i: (0, 0))],
        out_specs=[pl.BlockSpec((_BN, _CLS), lambda i: (i, 0)), row],
        out_shape=[jax.ShapeDtypeStruct((_N, _CLS), f32),
                   jax.ShapeDtypeStruct((_N, _D), f32)],
    )(pin0, pin1, pout0, pout1, sh, bmi, bmo, bsi, bso, cin, cout, wdec, bdec)


def _sc_body(hmi0, hmi1, hmo0, hmo1, ei_in, wi, ei_out, wo,
             pin0, pin1, pout0, pout1,
             acc_in, acc_out, eib0, eib1, dstb0, dstb1, wvb0, wvb1,
             rows0, rows1, zbuf,
             sei0, sei1, swm0, swm1, sg0, sg1, ss0, ss1):
    c = lax.axis_index("c")
    s = lax.axis_index("s")
    ebs = (eib0, eib1)
    dbs = (dstb0, dstb1)
    wbs = (wvb0, wvb1)
    rbs = (rows0, rows1)
    sei = (sei0, sei1)
    swm = (swm0, swm1)
    sg = (sg0, sg1)
    ss = (ss0, ss1)

    zeros16 = jnp.zeros((16,), jnp.float32)

    def zinit(i, carry):
        for k4 in range(_H // 16):
            zbuf[i, k4 * 16:(k4 + 1) * 16] = zeros16
        return carry

    lax.fori_loop(0, _ZR, zinit, 0)
    base_r = s * _RPT
    for kk in range(_RPT // _ZR):
        pltpu.sync_copy(zbuf, acc_in.at[pl.ds(base_r + kk * _ZR, _ZR)])
        pltpu.sync_copy(zbuf, acc_out.at[pl.ds(base_r + kk * _ZR, _ZR)])
    plsc.subcore_barrier()

    def run_edges(ei_h, w_h, tab_h, acc):
        def cbase(j):
            return jnp.minimum(j, _NCH - 1) * _C

        def idx_issue(j, b):
            pltpu.async_copy(ei_h.at[:, pl.ds(cbase(j), _C)], ebs[b], sei[b])

        def idx_wait(b):
            pltpu.make_async_copy(ei_h.at[:, pl.ds(0, _C)], ebs[b], sei[b]).wait()

        def w_issue(j, b):
            pltpu.async_copy(w_h.at[pl.ds(cbase(j), _C)], wbs[b], swm[b])

        def w_wait(b):
            pltpu.make_async_copy(w_h.at[pl.ds(0, _C)], wbs[b], swm[b]).wait()

        def gather_issue(b):
            pltpu.async_copy(tab_h.at[ebs[b].at[0]], rbs[b], sg[b])

        def gather_wait(b):
            pltpu.make_async_copy(tab_h.at[ebs[b].at[0]], rbs[b], sg[b]).wait()

        def scatter_issue(b):
            pltpu.async_copy(rbs[b], acc.at[dbs[b]], ss[b], add=True)

        def scatter_wait(b):
            pltpu.make_async_copy(rbs[b], acc.at[dbs[b]], ss[b]).wait()

        idx_issue(s, 0)
        idx_issue(_NS + s, 1)
        w_issue(s, 0)
        idx_wait(0)
        gather_issue(0)

        def pair(i, carry):
            for b in (0, 1):
                o = b ^ 1
                jj = 2 * i + b
                j = jj * _NS + s
                gather_wait(b)
                for g in range(_C // 16):
                    sl = pl.ds(g * 16, 16)
                    dbs[b][sl] = ebs[b][1, sl]
                idx_issue(j + 2 * _NS, b)
                w_issue(j + _NS, o)
                w_wait(b)
                validf = jnp.where(j < _NCH, 1.0, 0.0).astype(jnp.float32)

                def scale(g, cc):
                    w16 = wbs[b][pl.ds(g * 16, 16)] * validf
                    for e in range(16):
                        wsc = w16[e]
                        r = g * 16 + e
                        for k4 in range(_H // 16):
                            sl = pl.ds(k4 * 16, 16)
                            rbs[b][r, sl] = rbs[b][r, sl] * wsc
                    return cc

                lax.fori_loop(0, _C // 16, scale, 0, unroll=True)
                if b == 0:
                    @pl.when(i > 0)
                    def _():
                        scatter_wait(1)
                else:
                    scatter_wait(0)
                idx_wait(o)
                gather_issue(o)
                scatter_issue(b)
            return carry

        lax.fori_loop(0, (_T + 2) // 2, pair, 0)
        scatter_wait(1)
        gather_wait(0)
        idx_wait(1)
        w_wait(0)

    @pl.when(c == 0)
    def _():
        run_edges(ei_in, wi, hmi0, acc_in)
        run_edges(ei_out, wo, hmo0, acc_out)

    @pl.when(c == 1)
    def _():
        run_edges(ei_in, wi, hmi1, acc_in)
        run_edges(ei_out, wo, hmo1, acc_out)

    plsc.subcore_barrier()

    @pl.when(c == 0)
    def _():
        pltpu.sync_copy(acc_in.at[pl.ds(base_r, _RPT)], pin0.at[pl.ds(base_r, _RPT)])
        pltpu.sync_copy(acc_out.at[pl.ds(base_r, _RPT)], pout0.at[pl.ds(base_r, _RPT)])

    @pl.when(c == 1)
    def _():
        pltpu.sync_copy(acc_in.at[pl.ds(base_r, _RPT)], pin1.at[pl.ds(base_r, _RPT)])
        pltpu.sync_copy(acc_out.at[pl.ds(base_r, _RPT)], pout1.at[pl.ds(base_r, _RPT)])


def _sc_propagate(hmi0, hmi1, hmo0, hmo1, ei_in, wi, ei_out, wo):
    f32 = jnp.float32
    i32 = jnp.int32
    mesh = plsc.VectorSubcoreMesh(core_axis_name="c", subcore_axis_name="s")
    kfn = pl.kernel(
        _sc_body,
        out_type=[jax.ShapeDtypeStruct((_N, _H), f32)] * 4,
        mesh=mesh,
        compiler_params=pltpu.CompilerParams(use_tc_tiling_on_sc=False),
        scratch_types=[
            pltpu.VMEM_SHARED((_N, _H), f32),
            pltpu.VMEM_SHARED((_N, _H), f32),
            pltpu.VMEM((2, _C), i32),
            pltpu.VMEM((2, _C), i32),
            pltpu.VMEM((_C,), i32),
            pltpu.VMEM((_C,), i32),
            pltpu.VMEM((_C,), f32),
            pltpu.VMEM((_C,), f32),
            pltpu.VMEM((_C, _H), f32),
            pltpu.VMEM((_C, _H), f32),
            pltpu.VMEM((_ZR, _H), f32),
        ] + [pltpu.SemaphoreType.DMA] * 8,
    )
    return kfn(hmi0, hmi1, hmo0, hmo1, ei_in, wi, ei_out, wo)


def kernel(x, edge_index_in, edge_weight_in, edge_index_out, edge_weight_out,
           Wmi0, Wmo0, Ws0, bmi0, bmo0, bsi0, bso0, Cin0, Cout0,
           Wmi1, Wmo1, Ws1, bmi1, bmo1, bsi1, bso1, Cin1, Cout1,
           W_dec, b_dec):
    bmi0r, bmo0r = bmi0.reshape(1, _D), bmo0.reshape(1, _D)
    bsi0r, bso0r = bsi0.reshape(1, _D), bso0.reshape(1, _D)
    bmi1r, bmo1r = bmi1.reshape(1, _D), bmo1.reshape(1, _D)
    bsi1r, bso1r = bsi1.reshape(1, _D), bso1.reshape(1, _D)
    bdecr = b_dec.reshape(1, _CLS)

    hmi0a, hmi0b, hmo0a, hmo0b, sh0 = _tc_pre(x, Wmi0, Wmo0, Ws0)
    pin0a, pin0b, pout0a, pout0b = _sc_propagate(
        hmi0a, hmi0b, hmo0a, hmo0b, edge_index_in, edge_weight_in,
        edge_index_out, edge_weight_out)
    hmi1a, hmi1b, hmo1a, hmo1b, sh1 = _tc_mid(
        pin0a, pin0b, pout0a, pout0b, sh0, bmi0r, bmo0r, bsi0r, bso0r,
        Cin0, Cout0, Wmi1, Wmo1, Ws1)
    pin1a, pin1b, pout1a, pout1b = _sc_propagate(
        hmi1a, hmi1b, hmo1a, hmo1b, edge_index_in, edge_weight_in,
        edge_index_out, edge_weight_out)
    logp, emb = _tc_post(
        pin1a, pin1b, pout1a, pout1b, sh1, bmi1r, bmo1r, bsi1r, bso1r,
        Cin1, Cout1, W_dec, bdecr)
    return (logp, emb)


# speedup vs baseline: 6.3792x; 1.2173x over previous
import jax
import jax.numpy as jnp
from jax import lax
from jax.experimental import pallas as pl
from jax.experimental.pallas import tpu as pltpu
from jax.experimental.pallas import tpu_sc as plsc

_N = 10000
_D = 128
_H = 64
_E = 320000
_CLS = 10
_EPS = 1e-12

_C = 128
_NCH = _E // _C
_NS = 16
_T = -(-_NCH // _NS)
_RPT = _N // _NS
_ZR = 125

_BN = 1000
_NB = _N // _BN


_DN = (((1,), (1,)), ((), ()))


def _dot_t(a, b):
    return lax.dot_general(a, b, _DN, preferred_element_type=jnp.float32)


def _tc_pre_body(x_ref, wmi_ref, wmo_ref, ws_ref,
                 hmi0_ref, hmi1_ref, hmo0_ref, hmo1_ref, sh_ref):
    xb = x_ref[...]
    hmi0_ref[...] = _dot_t(xb, wmi_ref[0:_H, :])
    hmi1_ref[...] = _dot_t(xb, wmi_ref[_H:_D, :])
    hmo0_ref[...] = _dot_t(xb, wmo_ref[0:_H, :])
    hmo1_ref[...] = _dot_t(xb, wmo_ref[_H:_D, :])
    sh_ref[...] = _dot_t(xb, ws_ref[...])


def _tc_pre(x, wmi, wmo, ws):
    f32 = jnp.float32
    row = pl.BlockSpec((_BN, _D), lambda i: (i, 0))
    half = pl.BlockSpec((_BN, _H), lambda i: (i, 0))
    full_w = pl.BlockSpec((_D, _D), lambda i: (0, 0))
    return pl.pallas_call(
        _tc_pre_body,
        grid=(_NB,),
        in_specs=[row, full_w, full_w, full_w],
        out_specs=[half, half, half, half, row],
        out_shape=[jax.ShapeDtypeStruct((_N, _H), f32)] * 4
        + [jax.ShapeDtypeStruct((_N, _D), f32)],
    )(x, wmi, wmo, ws)


def _combine(pin0, pin1, pout0, pout1, sh_ref, bmi, bmo, bsi, bso, cin, cout):
    sh = sh_ref[...]
    pin = jnp.concatenate([pin0[...], pin1[...]], axis=1)
    pout = jnp.concatenate([pout0[...], pout1[...]], axis=1)
    ic = pin + bmi[...] + sh + bsi[...]
    oc = pout + bmo[...] + sh + bso[...]
    return jnp.tanh(cin[...] * ic + cout[...] * oc)


def _tc_mid_body(pin0, pin1, pout0, pout1, sh_ref, bmi, bmo, bsi, bso,
                 cin, cout, wmi_ref, wmo_ref, ws_ref,
                 hmi0_ref, hmi1_ref, hmo0_ref, hmo1_ref, sh_o):
    h = _combine(pin0, pin1, pout0, pout1, sh_ref, bmi, bmo, bsi, bso, cin, cout)
    hmi0_ref[...] = _dot_t(h, wmi_ref[0:_H, :])
    hmi1_ref[...] = _dot_t(h, wmi_ref[_H:_D, :])
    hmo0_ref[...] = _dot_t(h, wmo_ref[0:_H, :])
    hmo1_ref[...] = _dot_t(h, wmo_ref[_H:_D, :])
    sh_o[...] = _dot_t(h, ws_ref[...])


def _tc_mid(pin0, pin1, pout0, pout1, sh, bmi, bmo, bsi, bso, cin, cout,
            wmi, wmo, ws):
    f32 = jnp.float32
    row = pl.BlockSpec((_BN, _D), lambda i: (i, 0))
    half = pl.BlockSpec((_BN, _H), lambda i: (i, 0))
    bias = pl.BlockSpec((1, _D), lambda i: (0, 0))
    cvec = pl.BlockSpec((_BN, 1), lambda i: (i, 0))
    full_w = pl.BlockSpec((_D, _D), lambda i: (0, 0))
    return pl.pallas_call(
        _tc_mid_body,
        grid=(_NB,),
        in_specs=[half, half, half, half, row, bias, bias, bias, bias,
                  cvec, cvec, full_w, full_w, full_w],
        out_specs=[half, half, half, half, row],
        out_shape=[jax.ShapeDtypeStruct((_N, _H), f32)] * 4
        + [jax.ShapeDtypeStruct((_N, _D), f32)],
    )(pin0, pin1, pout0, pout1, sh, bmi, bmo, bsi, bso, cin, cout, wmi, wmo, ws)


def _tc_post_body(pin0, pin1, pout0, pout1, sh_ref, bmi, bmo, bsi, bso,
                  cin, cout, wdec_ref, bdec_ref, logp_ref, emb_ref):
    h = _combine(pin0, pin1, pout0, pout1, sh_ref, bmi, bmo, bsi, bso, cin, cout)
    logits = _dot_t(h, wdec_ref[...]) + bdec_ref[...]
    m = jnp.max(logits, axis=-1, keepdims=True)
    e = jnp.exp(logits - m)
    lse = jnp.log(jnp.sum(e, axis=-1, keepdims=True)) + m
    logp_ref[...] = logits - lse
    nrm = jnp.sqrt(jnp.sum(h * h, axis=-1, keepdims=True))
    emb_ref[...] = h / (nrm + _EPS)


def _tc_post(pin0, pin1, pout0, pout1, sh, bmi, bmo, bsi, bso, cin, cout,
             wdec, bdec):
    f32 = jnp.float32
    row = pl.BlockSpec((_BN, _D), lambda i: (i, 0))
    half = pl.BlockSpec((_BN, _H), lambda i: (i, 0))
    bias = pl.BlockSpec((1, _D), lambda i: (0, 0))
    cvec = pl.BlockSpec((_BN, 1), lambda i: (i, 0))
    return pl.pallas_call(
        _tc_post_body,
        grid=(_NB,),
        in_specs=[half, half, half, half, row, bias, bias, bias, bias,
                  cvec, cvec,
                  pl.BlockSpec((_CLS, _D), lambda i: (0, 0)),
                  pl.BlockSpec((1, _CLS), lambda i: (0, 0))],
        out_specs=[pl.BlockSpec((_BN, _CLS), lambda i: (i, 0)), row],
        out_shape=[jax.ShapeDtypeStruct((_N, _CLS), f32),
                   jax.ShapeDtypeStruct((_N, _D), f32)],
    )(pin0, pin1, pout0, pout1, sh, bmi, bmo, bsi, bso, cin, cout, wdec, bdec)


def _sc_body(hmi0, hmi1, hmo0, hmo1, ei_in, wi, ei_out, wo,
             pin0, pin1, pout0, pout1,
             acc, tab_s, eib0, eib1, dstb0, dstb1, wvb0, wvb1,
             rows0, rows1, zbuf,
             sei0, sei1, swm0, swm1, sg0, sg1, ss0, ss1):
    c = lax.axis_index("c")
    s = lax.axis_index("s")
    ebs = (eib0, eib1)
    dbs = (dstb0, dstb1)
    wbs = (wvb0, wvb1)
    rbs = (rows0, rows1)
    sei = (sei0, sei1)
    swm = (swm0, swm1)
    sg = (sg0, sg1)
    ss = (ss0, ss1)

    zeros16 = jnp.zeros((16,), jnp.float32)

    def zinit(i, carry):
        for k4 in range(_H // 16):
            zbuf[i, k4 * 16:(k4 + 1) * 16] = zeros16
        return carry

    lax.fori_loop(0, _ZR, zinit, 0)
    base_r = s * _RPT

    def zero_acc():
        for kk in range(_RPT // _ZR):
            pltpu.sync_copy(zbuf, acc.at[pl.ds(base_r + kk * _ZR, _ZR)])

    def run_edges(ei_h, w_h, tab_h):
        pltpu.sync_copy(tab_h.at[pl.ds(base_r, _RPT)],
                        tab_s.at[pl.ds(base_r, _RPT)])
        plsc.subcore_barrier()
        def cbase(j):
            return jnp.minimum(j, _NCH - 1) * _C

        def idx_issue(j, b):
            pltpu.async_copy(ei_h.at[:, pl.ds(cbase(j), _C)], ebs[b], sei[b])

        def idx_wait(b):
            pltpu.make_async_copy(ei_h.at[:, pl.ds(0, _C)], ebs[b], sei[b]).wait()

        def w_issue(j, b):
            pltpu.async_copy(w_h.at[pl.ds(cbase(j), _C)], wbs[b], swm[b])

        def w_wait(b):
            pltpu.make_async_copy(w_h.at[pl.ds(0, _C)], wbs[b], swm[b]).wait()

        def gather_issue(b):
            pltpu.async_copy(tab_s.at[ebs[b].at[0]], rbs[b], sg[b])

        def gather_wait(b):
            pltpu.make_async_copy(tab_s.at[ebs[b].at[0]], rbs[b], sg[b]).wait()

        def scatter_issue(b):
            pltpu.async_copy(rbs[b], acc.at[dbs[b]], ss[b], add=True)

        def scatter_wait(b):
            pltpu.make_async_copy(rbs[b], acc.at[dbs[b]], ss[b]).wait()

        idx_issue(s, 0)
        idx_issue(_NS + s, 1)
        w_issue(s, 0)
        idx_wait(0)
        gather_issue(0)

        def pair(i, carry):
            for b in (0, 1):
                o = b ^ 1
                jj = 2 * i + b
                j = jj * _NS + s
                gather_wait(b)
                for g in range(_C // 16):
                    sl = pl.ds(g * 16, 16)
                    dbs[b][sl] = ebs[b][1, sl]
                idx_issue(j + 2 * _NS, b)
                w_issue(j + _NS, o)
                w_wait(b)
                validf = jnp.where(j < _NCH, 1.0, 0.0).astype(jnp.float32)

                def scale(g, cc):
                    w16 = wbs[b][pl.ds(g * 16, 16)] * validf
                    for e in range(16):
                        wsc = w16[e]
                        r = g * 16 + e
                        for k4 in range(_H // 16):
                            sl = pl.ds(k4 * 16, 16)
                            rbs[b][r, sl] = rbs[b][r, sl] * wsc
                    return cc

                lax.fori_loop(0, _C // 16, scale, 0, unroll=True)
                if b == 0:
                    @pl.when(i > 0)
                    def _():
                        scatter_wait(1)
                else:
                    scatter_wait(0)
                idx_wait(o)
                gather_issue(o)
                scatter_issue(b)
            return carry

        lax.fori_loop(0, (_T + 2) // 2, pair, 0)
        scatter_wait(1)
        gather_wait(0)
        idx_wait(1)
        w_wait(0)
        plsc.subcore_barrier()

    zero_acc()

    @pl.when(c == 0)
    def _():
        run_edges(ei_in, wi, hmi0)
        pltpu.sync_copy(acc.at[pl.ds(base_r, _RPT)], pin0.at[pl.ds(base_r, _RPT)])
        zero_acc()
        run_edges(ei_out, wo, hmo0)
        pltpu.sync_copy(acc.at[pl.ds(base_r, _RPT)], pout0.at[pl.ds(base_r, _RPT)])

    @pl.when(c == 1)
    def _():
        run_edges(ei_in, wi, hmi1)
        pltpu.sync_copy(acc.at[pl.ds(base_r, _RPT)], pin1.at[pl.ds(base_r, _RPT)])
        zero_acc()
        run_edges(ei_out, wo, hmo1)
        pltpu.sync_copy(acc.at[pl.ds(base_r, _RPT)], pout1.at[pl.ds(base_r, _RPT)])


def _sc_propagate(hmi0, hmi1, hmo0, hmo1, ei_in, wi, ei_out, wo):
    f32 = jnp.float32
    i32 = jnp.int32
    mesh = plsc.VectorSubcoreMesh(core_axis_name="c", subcore_axis_name="s")
    kfn = pl.kernel(
        _sc_body,
        out_type=[jax.ShapeDtypeStruct((_N, _H), f32)] * 4,
        mesh=mesh,
        compiler_params=pltpu.CompilerParams(use_tc_tiling_on_sc=False),
        scratch_types=[
            pltpu.VMEM_SHARED((_N, _H), f32),
            pltpu.VMEM_SHARED((_N, _H), f32),
            pltpu.VMEM((2, _C), i32),
            pltpu.VMEM((2, _C), i32),
            pltpu.VMEM((_C,), i32),
            pltpu.VMEM((_C,), i32),
            pltpu.VMEM((_C,), f32),
            pltpu.VMEM((_C,), f32),
            pltpu.VMEM((_C, _H), f32),
            pltpu.VMEM((_C, _H), f32),
            pltpu.VMEM((_ZR, _H), f32),
        ] + [pltpu.SemaphoreType.DMA] * 8,
    )
    return kfn(hmi0, hmi1, hmo0, hmo1, ei_in, wi, ei_out, wo)


def kernel(x, edge_index_in, edge_weight_in, edge_index_out, edge_weight_out,
           Wmi0, Wmo0, Ws0, bmi0, bmo0, bsi0, bso0, Cin0, Cout0,
           Wmi1, Wmo1, Ws1, bmi1, bmo1, bsi1, bso1, Cin1, Cout1,
           W_dec, b_dec):
    bmi0r, bmo0r = bmi0.reshape(1, _D), bmo0.reshape(1, _D)
    bsi0r, bso0r = bsi0.reshape(1, _D), bso0.reshape(1, _D)
    bmi1r, bmo1r = bmi1.reshape(1, _D), bmo1.reshape(1, _D)
    bsi1r, bso1r = bsi1.reshape(1, _D), bso1.reshape(1, _D)
    bdecr = b_dec.reshape(1, _CLS)

    hmi0a, hmi0b, hmo0a, hmo0b, sh0 = _tc_pre(x, Wmi0, Wmo0, Ws0)
    pin0a, pin0b, pout0a, pout0b = _sc_propagate(
        hmi0a, hmi0b, hmo0a, hmo0b, edge_index_in, edge_weight_in,
        edge_index_out, edge_weight_out)
    hmi1a, hmi1b, hmo1a, hmo1b, sh1 = _tc_mid(
        pin0a, pin0b, pout0a, pout0b, sh0, bmi0r, bmo0r, bsi0r, bso0r,
        Cin0, Cout0, Wmi1, Wmo1, Ws1)
    pin1a, pin1b, pout1a, pout1b = _sc_propagate(
        hmi1a, hmi1b, hmo1a, hmo1b, edge_index_in, edge_weight_in,
        edge_index_out, edge_weight_out)
    logp, emb = _tc_post(
        pin1a, pin1b, pout1a, pout1b, sh1, bmi1r, bmo1r, bsi1r, bso1r,
        Cin1, Cout1, W_dec, bdecr)
    return (logp, emb)
